# Initial kernel scaffold; baseline (speedup 1.0000x reference)
import functools

import jax
import jax.numpy as jnp
from jax import lax
from jax.experimental import pallas as pl
from jax.experimental.pallas import tpu as pltpu
from jax.experimental.pallas import tpu_sc as plsc

_NC = 2
_NS = 16
_NW = _NC * _NS
_LANES = 16

_CHUNK = 40


def _tc_pre_body(nf_ref, ef_ref, w1a_ref, w1b_ref, w1c_ref, b1_ref,
                 p0_ref, p1_ref, e_ref):
    nf = nf_ref[...]
    p0_ref[...] = jnp.dot(nf, w1a_ref[...], preferred_element_type=jnp.float32)
    p1_ref[...] = jnp.dot(nf, w1b_ref[...], preferred_element_type=jnp.float32)
    e_ref[...] = jnp.dot(ef_ref[...], w1c_ref[...],
                         preferred_element_type=jnp.float32) + b1_ref[...]


def _tc_post_body(nf_ref, s0_ref, s1_ref, w2a_ref, w2b_ref, b2_ref, out_ref):
    acc = jnp.dot(nf_ref[...], w2a_ref[...], preferred_element_type=jnp.float32)
    s = s0_ref[...] + s1_ref[...]
    acc = acc + jnp.dot(s, w2b_ref[...], preferred_element_type=jnp.float32)
    out_ref[...] = jnp.maximum(acc + b2_ref[...], 0.0)


def _sc_edge_body(n_nodes, n_edges,
                  p0_hbm, p1_hbm, e_hbm, n0_hbm, n1_hbm, out_hbm,
                  idx0, idx1, r0, r1, re, zbuf, acc_sh, sem0, sem1, sem2):
    cid = lax.axis_index("c")
    sid = lax.axis_index("s")
    wid = sid * _NC + cid

    rows_per_tile = n_nodes // _NS
    zeros16 = jnp.zeros((_LANES,), jnp.float32)

    def zero_body(j, carry):
        zbuf[j, pl.ds(0, _LANES)] = zeros16
        zbuf[j, pl.ds(_LANES, _LANES)] = zeros16
        return carry

    lax.fori_loop(0, rows_per_tile, zero_body, 0)
    pltpu.sync_copy(zbuf, acc_sh.at[pl.ds(sid * rows_per_tile, rows_per_tile)])
    plsc.subcore_barrier()

    edges_per_tile = n_edges // _NW
    tile_base = wid * edges_per_tile
    nchunks = edges_per_tile // _CHUNK

    def chunk_body(c, carry):
        base = tile_base + c * _CHUNK
        cp0 = pltpu.async_copy(n0_hbm.at[pl.ds(base, _CHUNK)], idx0, sem0)
        cp1 = pltpu.async_copy(n1_hbm.at[pl.ds(base, _CHUNK)], idx1, sem1)
        cp2 = pltpu.async_copy(e_hbm.at[pl.ds(base, _CHUNK)], re, sem2)
        cp0.wait()
        cp1.wait()
        g0 = pltpu.async_copy(p0_hbm.at[idx0], r0, sem0)
        g1 = pltpu.async_copy(p1_hbm.at[idx1], r1, sem1)
        cp2.wait()
        g0.wait()
        g1.wait()

        def row_body(j, inner_carry):
            for h in (0, _LANES):
                v = (r0[j, pl.ds(h, _LANES)] + r1[j, pl.ds(h, _LANES)]
                     + re[j, pl.ds(h, _LANES)])
                re[j, pl.ds(h, _LANES)] = jnp.maximum(v, 0.0)
            return inner_carry

        lax.fori_loop(0, _CHUNK, row_body, 0)
        pltpu.sync_copy(re, acc_sh.at[idx0], add=True)
        return carry

    lax.fori_loop(0, nchunks, chunk_body, 0)
    plsc.subcore_barrier()
    pltpu.sync_copy(
        acc_sh.at[pl.ds(sid * rows_per_tile, rows_per_tile)],
        out_hbm.at[cid, pl.ds(sid * rows_per_tile, rows_per_tile)])


def kernel(node_features, edge_node_indices, edge_features, W1, b1, W2, b2):
    n_nodes, d_feat = node_features.shape
    n_edges = edge_features.shape[0]
    msg = W1.shape[1]
    d_edge = edge_features.shape[1]

    n0 = edge_node_indices[0]
    n1 = edge_node_indices[1]
    w1a = W1[:d_feat]
    w1b = W1[d_feat:2 * d_feat]
    w1c = W1[2 * d_feat:]
    b1r = b1.reshape(1, msg)
    b2r = b2.reshape(1, -1)

    grid1 = 20
    nblk = n_nodes // grid1
    eblk = n_edges // grid1
    p0, p1, e_pre = pl.pallas_call(
        _tc_pre_body,
        grid=(grid1,),
        in_specs=[
            pl.BlockSpec((nblk, d_feat), lambda i: (i, 0)),
            pl.BlockSpec((eblk, d_edge), lambda i: (i, 0)),
            pl.BlockSpec((d_feat, msg), lambda i: (0, 0)),
            pl.BlockSpec((d_feat, msg), lambda i: (0, 0)),
            pl.BlockSpec((d_edge, msg), lambda i: (0, 0)),
            pl.BlockSpec((1, msg), lambda i: (0, 0)),
        ],
        out_specs=[
            pl.BlockSpec((nblk, msg), lambda i: (i, 0)),
            pl.BlockSpec((nblk, msg), lambda i: (i, 0)),
            pl.BlockSpec((eblk, msg), lambda i: (i, 0)),
        ],
        out_shape=[
            jax.ShapeDtypeStruct((n_nodes, msg), jnp.float32),
            jax.ShapeDtypeStruct((n_nodes, msg), jnp.float32),
            jax.ShapeDtypeStruct((n_edges, msg), jnp.float32),
        ],
    )(node_features, edge_features, w1a, w1b, w1c, b1r)

    mesh = plsc.VectorSubcoreMesh(core_axis_name="c", subcore_axis_name="s")
    sc_fn = pl.kernel(
        functools.partial(_sc_edge_body, n_nodes, n_edges),
        mesh=mesh,
        out_type=jax.ShapeDtypeStruct((_NC, n_nodes, msg), jnp.float32),
        scratch_types=[
            pltpu.VMEM((_CHUNK,), jnp.int32),
            pltpu.VMEM((_CHUNK,), jnp.int32),
            pltpu.VMEM((_CHUNK, msg), jnp.float32),
            pltpu.VMEM((_CHUNK, msg), jnp.float32),
            pltpu.VMEM((_CHUNK, msg), jnp.float32),
            pltpu.VMEM((n_nodes // _NS, msg), jnp.float32),
            pltpu.VMEM_SHARED((n_nodes, msg), jnp.float32),
            pltpu.SemaphoreType.DMA,
            pltpu.SemaphoreType.DMA,
            pltpu.SemaphoreType.DMA,
        ],
    )
    partials = sc_fn(p0, p1, e_pre, n0, n1)

    grid2 = 10
    nblk2 = n_nodes // grid2
    out = pl.pallas_call(
        _tc_post_body,
        grid=(grid2,),
        in_specs=[
            pl.BlockSpec((nblk2, d_feat), lambda i: (i, 0)),
            pl.BlockSpec((nblk2, msg), lambda i: (i, 0)),
            pl.BlockSpec((nblk2, msg), lambda i: (i, 0)),
            pl.BlockSpec((d_feat, d_feat), lambda i: (0, 0)),
            pl.BlockSpec((msg, d_feat), lambda i: (0, 0)),
            pl.BlockSpec((1, d_feat), lambda i: (0, 0)),
        ],
        out_specs=pl.BlockSpec((nblk2, d_feat), lambda i: (i, 0)),
        out_shape=jax.ShapeDtypeStruct((n_nodes, d_feat), jnp.float32),
    )(node_features, partials[0], partials[1], W2[:d_feat], W2[d_feat:], b2r)

    return out


# speedup vs baseline: 3.9525x; 3.9525x over previous
import functools

import jax
import jax.numpy as jnp
from jax import lax
from jax.experimental import pallas as pl
from jax.experimental.pallas import tpu as pltpu
from jax.experimental.pallas import tpu_sc as plsc

_NC = 2
_NS = 16
_NW = _NC * _NS
_LANES = 16

_CHUNK = 40


def _tc_pre_body(nf_ref, ef_ref, w1a_ref, w1b_ref, w1c_ref, b1_ref,
                 p0_ref, p1_ref, e_ref):
    nf = nf_ref[...]
    p0_ref[...] = jnp.dot(nf, w1a_ref[...], preferred_element_type=jnp.float32)
    p1_ref[...] = jnp.dot(nf, w1b_ref[...], preferred_element_type=jnp.float32)
    e_ref[...] = jnp.dot(ef_ref[...], w1c_ref[...],
                         preferred_element_type=jnp.float32) + b1_ref[...]


def _tc_post_body(nf_ref, s0_ref, s1_ref, w2a_ref, w2b_ref, b2_ref, out_ref):
    acc = jnp.dot(nf_ref[...], w2a_ref[...], preferred_element_type=jnp.float32)
    s = s0_ref[...] + s1_ref[...]
    acc = acc + jnp.dot(s, w2b_ref[...], preferred_element_type=jnp.float32)
    out_ref[...] = jnp.maximum(acc + b2_ref[...], 0.0)


def _sc_edge_body(n_nodes_pad, n_edges,
                  p0_hbm, p1_hbm, e_hbm, n0_hbm, n1_hbm, out_hbm,
                  idx0, idx1, r0, r1, re, zbuf, acc_sh, sem0, sem1, sem2):
    cid = lax.axis_index("c")
    sid = lax.axis_index("s")
    wid = sid * _NC + cid

    rows_per_tile = n_nodes_pad // _NS
    zeros16 = jnp.zeros((_LANES,), jnp.float32)

    def zero_body(j, carry):
        zbuf[j, pl.ds(0, _LANES)] = zeros16
        zbuf[j, pl.ds(_LANES, _LANES)] = zeros16
        return carry

    lax.fori_loop(0, rows_per_tile, zero_body, 0)
    pltpu.sync_copy(zbuf, acc_sh.at[pl.ds(sid * rows_per_tile, rows_per_tile)])
    plsc.subcore_barrier()

    edges_per_tile = n_edges // _NW
    tile_base = wid * edges_per_tile
    nchunks = edges_per_tile // _CHUNK

    def chunk_body(c, carry):
        base = tile_base + c * _CHUNK
        cp0 = pltpu.async_copy(n0_hbm.at[pl.ds(base, _CHUNK)], idx0, sem0)
        cp1 = pltpu.async_copy(n1_hbm.at[pl.ds(base, _CHUNK)], idx1, sem1)
        cp2 = pltpu.async_copy(e_hbm.at[pl.ds(base, _CHUNK)], re, sem2)
        cp0.wait()
        cp1.wait()
        g0 = pltpu.async_copy(p0_hbm.at[idx0], r0, sem0)
        g1 = pltpu.async_copy(p1_hbm.at[idx1], r1, sem1)
        cp2.wait()
        g0.wait()
        g1.wait()

        def row_body(j, inner_carry):
            for h in (0, _LANES):
                v = (r0[j, pl.ds(h, _LANES)] + r1[j, pl.ds(h, _LANES)]
                     + re[j, pl.ds(h, _LANES)])
                re[j, pl.ds(h, _LANES)] = jnp.maximum(v, 0.0)
            return inner_carry

        lax.fori_loop(0, _CHUNK, row_body, 0)
        pltpu.sync_copy(re, acc_sh.at[idx0], add=True)
        return carry

    lax.fori_loop(0, nchunks, chunk_body, 0)
    plsc.subcore_barrier()
    pltpu.sync_copy(
        acc_sh.at[pl.ds(sid * rows_per_tile, rows_per_tile)],
        out_hbm.at[cid, pl.ds(sid * rows_per_tile, rows_per_tile)])


def kernel(node_features, edge_node_indices, edge_features, W1, b1, W2, b2):
    n_nodes, d_feat = node_features.shape
    n_edges = edge_features.shape[0]
    msg = W1.shape[1]
    d_edge = edge_features.shape[1]

    n0 = edge_node_indices[0]
    n1 = edge_node_indices[1]
    w1a = W1[:d_feat]
    w1b = W1[d_feat:2 * d_feat]
    w1c = W1[2 * d_feat:]
    b1r = b1.reshape(1, msg)
    b2r = b2.reshape(1, -1)

    grid1 = 10
    nblk = n_nodes // grid1
    eblk = n_edges // grid1
    p0, p1, e_pre = pl.pallas_call(
        _tc_pre_body,
        grid=(grid1,),
        in_specs=[
            pl.BlockSpec((nblk, d_feat), lambda i: (i, 0)),
            pl.BlockSpec((eblk, d_edge), lambda i: (i, 0)),
            pl.BlockSpec((d_feat, msg), lambda i: (0, 0)),
            pl.BlockSpec((d_feat, msg), lambda i: (0, 0)),
            pl.BlockSpec((d_edge, msg), lambda i: (0, 0)),
            pl.BlockSpec((1, msg), lambda i: (0, 0)),
        ],
        out_specs=[
            pl.BlockSpec((nblk, msg), lambda i: (i, 0)),
            pl.BlockSpec((nblk, msg), lambda i: (i, 0)),
            pl.BlockSpec((eblk, msg), lambda i: (i, 0)),
        ],
        out_shape=[
            jax.ShapeDtypeStruct((n_nodes, msg), jnp.float32),
            jax.ShapeDtypeStruct((n_nodes, msg), jnp.float32),
            jax.ShapeDtypeStruct((n_edges, msg), jnp.float32),
        ],
    )(node_features, edge_features, w1a, w1b, w1c, b1r)

    n_nodes_pad = ((n_nodes + 8 * _NS - 1) // (8 * _NS)) * (8 * _NS)
    mesh = plsc.VectorSubcoreMesh(core_axis_name="c", subcore_axis_name="s")
    sc_fn = pl.kernel(
        functools.partial(_sc_edge_body, n_nodes_pad, n_edges),
        mesh=mesh,
        compiler_params=pltpu.CompilerParams(use_tc_tiling_on_sc=False),
        out_type=jax.ShapeDtypeStruct((_NC, n_nodes_pad, msg), jnp.float32),
        scratch_types=[
            pltpu.VMEM((_CHUNK,), jnp.int32),
            pltpu.VMEM((_CHUNK,), jnp.int32),
            pltpu.VMEM((_CHUNK, msg), jnp.float32),
            pltpu.VMEM((_CHUNK, msg), jnp.float32),
            pltpu.VMEM((_CHUNK, msg), jnp.float32),
            pltpu.VMEM((n_nodes_pad // _NS, msg), jnp.float32),
            pltpu.VMEM_SHARED((n_nodes_pad, msg), jnp.float32),
            pltpu.SemaphoreType.DMA,
            pltpu.SemaphoreType.DMA,
            pltpu.SemaphoreType.DMA,
        ],
    )
    partials = sc_fn(p0, p1, e_pre, n0, n1)

    grid2 = 10
    nblk2 = n_nodes // grid2
    out = pl.pallas_call(
        _tc_post_body,
        grid=(grid2,),
        in_specs=[
            pl.BlockSpec((nblk2, d_feat), lambda i: (i, 0)),
            pl.BlockSpec((nblk2, msg), lambda i: (i, 0)),
            pl.BlockSpec((nblk2, msg), lambda i: (i, 0)),
            pl.BlockSpec((d_feat, d_feat), lambda i: (0, 0)),
            pl.BlockSpec((msg, d_feat), lambda i: (0, 0)),
            pl.BlockSpec((1, d_feat), lambda i: (0, 0)),
        ],
        out_specs=pl.BlockSpec((nblk2, d_feat), lambda i: (i, 0)),
        out_shape=jax.ShapeDtypeStruct((n_nodes, d_feat), jnp.float32),
    )(node_features, partials[0], partials[1], W2[:d_feat], W2[d_feat:], b2r)

    return out


# speedup vs baseline: 4.8607x; 1.2298x over previous
import functools

import jax
import jax.numpy as jnp
from jax import lax
from jax.experimental import pallas as pl
from jax.experimental.pallas import tpu as pltpu
from jax.experimental.pallas import tpu_sc as plsc

_NC = 2
_NS = 16
_NW = _NC * _NS
_LANES = 16

_CHUNK = 128


def _tc_pre_body(nf_ref, ef_ref, w1a_ref, w1b_ref, w1c_ref, b1_ref,
                 p0_ref, p1_ref, e_ref):
    nf = nf_ref[...]
    p0_ref[...] = jnp.dot(nf, w1a_ref[...], preferred_element_type=jnp.float32)
    p1_ref[...] = jnp.dot(nf, w1b_ref[...], preferred_element_type=jnp.float32)
    e_ref[...] = jnp.dot(ef_ref[...], w1c_ref[...],
                         preferred_element_type=jnp.float32) + b1_ref[...]


def _tc_post_body(nf_ref, s0_ref, s1_ref, w2a_ref, w2b_ref, b2_ref, out_ref):
    acc = jnp.dot(nf_ref[...], w2a_ref[...], preferred_element_type=jnp.float32)
    s = s0_ref[...] + s1_ref[...]
    acc = acc + jnp.dot(s, w2b_ref[...], preferred_element_type=jnp.float32)
    out_ref[...] = jnp.maximum(acc + b2_ref[...], 0.0)


def _sc_edge_body(n_nodes_pad, n_edges,
                  p0_hbm, p1_hbm, e_hbm, n0_hbm, n1_hbm, out_hbm,
                  idx0a, idx0b, idx1a, idx1b, r0a, r0b, r1a, r1b, rea, reb,
                  zbuf, acc_sh,
                  si0a, si0b, si1a, si1b, sea, seb, sg0a, sg0b, sg1a, sg1b):
    cid = lax.axis_index("c")
    sid = lax.axis_index("s")
    wid = sid * _NC + cid

    idx0 = (idx0a, idx0b)
    idx1 = (idx1a, idx1b)
    r0 = (r0a, r0b)
    r1 = (r1a, r1b)
    re = (rea, reb)
    si0 = (si0a, si0b)
    si1 = (si1a, si1b)
    se = (sea, seb)
    sg0 = (sg0a, sg0b)
    sg1 = (sg1a, sg1b)

    total_chunks = n_edges // _CHUNK
    nbase = total_chunks // _NW
    rem = total_chunks % _NW
    nch = nbase + jnp.where(wid < rem, 1, 0)
    slots = nbase + (1 if rem else 0)
    groups = (slots + 1) // 2

    def chunk_base(c):
        return (wid + _NW * c) * _CHUNK

    def issue_idx_loads(c, b):
        base = chunk_base(c)
        pltpu.async_copy(n0_hbm.at[pl.ds(base, _CHUNK)], idx0[b], si0[b])
        pltpu.async_copy(n1_hbm.at[pl.ds(base, _CHUNK)], idx1[b], si1[b])

    def wait_idx_loads(b):
        pltpu.make_async_copy(n0_hbm.at[pl.ds(0, _CHUNK)], idx0[b], si0[b]).wait()
        pltpu.make_async_copy(n1_hbm.at[pl.ds(0, _CHUNK)], idx1[b], si1[b]).wait()

    def issue_e_load(c, b):
        base = chunk_base(c)
        pltpu.async_copy(e_hbm.at[pl.ds(base, _CHUNK)], re[b], se[b])

    def wait_e_load(b):
        pltpu.make_async_copy(e_hbm.at[pl.ds(0, _CHUNK)], re[b], se[b]).wait()

    def issue_gathers(b):
        pltpu.async_copy(p0_hbm.at[idx0[b]], r0[b], sg0[b])
        pltpu.async_copy(p1_hbm.at[idx1[b]], r1[b], sg1[b])

    def wait_gathers(b):
        pltpu.make_async_copy(p0_hbm.at[idx0[b]], r0[b], sg0[b]).wait()
        pltpu.make_async_copy(p1_hbm.at[idx1[b]], r1[b], sg1[b]).wait()

    issue_idx_loads(0, 0)
    issue_e_load(0, 0)
    issue_idx_loads(1, 1)
    issue_e_load(1, 1)

    rows_per_tile = n_nodes_pad // _NS
    zeros16 = jnp.zeros((_LANES,), jnp.float32)

    def zero_body(j, carry):
        zbuf[j, pl.ds(0, _LANES)] = zeros16
        zbuf[j, pl.ds(_LANES, _LANES)] = zeros16
        return carry

    lax.fori_loop(0, rows_per_tile, zero_body, 0)
    pltpu.sync_copy(zbuf, acc_sh.at[pl.ds(sid * rows_per_tile, rows_per_tile)])
    plsc.subcore_barrier()

    wait_idx_loads(0)
    issue_gathers(0)

    def group_body(g, carry):
        for k in (0, 1):
            c = g * 2 + k
            b = k
            nb = 1 - k

            @pl.when(c + 1 < nch)
            def _():
                wait_idx_loads(nb)
                issue_gathers(nb)

            @pl.when(c < nch)
            def _():
                wait_gathers(b)
                wait_e_load(b)

                def row_body(j, inner):
                    for h in (0, _LANES):
                        v = (r0[b][j, pl.ds(h, _LANES)]
                             + r1[b][j, pl.ds(h, _LANES)]
                             + re[b][j, pl.ds(h, _LANES)])
                        re[b][j, pl.ds(h, _LANES)] = jnp.maximum(v, 0.0)
                    return inner

                lax.fori_loop(0, _CHUNK, row_body, 0, unroll=8)
                pltpu.sync_copy(re[b], acc_sh.at[idx0[b]], add=True)

            @pl.when(c + 2 < nch)
            def _():
                issue_idx_loads(c + 2, b)
                issue_e_load(c + 2, b)
        return carry

    lax.fori_loop(0, groups, group_body, 0)

    plsc.subcore_barrier()
    pltpu.sync_copy(
        acc_sh.at[pl.ds(sid * rows_per_tile, rows_per_tile)],
        out_hbm.at[cid, pl.ds(sid * rows_per_tile, rows_per_tile)])


def kernel(node_features, edge_node_indices, edge_features, W1, b1, W2, b2):
    n_nodes, d_feat = node_features.shape
    n_edges = edge_features.shape[0]
    msg = W1.shape[1]
    d_edge = edge_features.shape[1]

    n0 = edge_node_indices[0]
    n1 = edge_node_indices[1]
    w1a = W1[:d_feat]
    w1b = W1[d_feat:2 * d_feat]
    w1c = W1[2 * d_feat:]
    b1r = b1.reshape(1, msg)
    b2r = b2.reshape(1, -1)

    grid1 = 10
    nblk = n_nodes // grid1
    eblk = n_edges // grid1
    p0, p1, e_pre = pl.pallas_call(
        _tc_pre_body,
        grid=(grid1,),
        in_specs=[
            pl.BlockSpec((nblk, d_feat), lambda i: (i, 0)),
            pl.BlockSpec((eblk, d_edge), lambda i: (i, 0)),
            pl.BlockSpec((d_feat, msg), lambda i: (0, 0)),
            pl.BlockSpec((d_feat, msg), lambda i: (0, 0)),
            pl.BlockSpec((d_edge, msg), lambda i: (0, 0)),
            pl.BlockSpec((1, msg), lambda i: (0, 0)),
        ],
        out_specs=[
            pl.BlockSpec((nblk, msg), lambda i: (i, 0)),
            pl.BlockSpec((nblk, msg), lambda i: (i, 0)),
            pl.BlockSpec((eblk, msg), lambda i: (i, 0)),
        ],
        out_shape=[
            jax.ShapeDtypeStruct((n_nodes, msg), jnp.float32),
            jax.ShapeDtypeStruct((n_nodes, msg), jnp.float32),
            jax.ShapeDtypeStruct((n_edges, msg), jnp.float32),
        ],
    )(node_features, edge_features, w1a, w1b, w1c, b1r)

    n_nodes_pad = ((n_nodes + 8 * _NS - 1) // (8 * _NS)) * (8 * _NS)
    mesh = plsc.VectorSubcoreMesh(core_axis_name="c", subcore_axis_name="s")
    sc_fn = pl.kernel(
        functools.partial(_sc_edge_body, n_nodes_pad, n_edges),
        mesh=mesh,
        compiler_params=pltpu.CompilerParams(use_tc_tiling_on_sc=False),
        out_type=jax.ShapeDtypeStruct((_NC, n_nodes_pad, msg), jnp.float32),
        scratch_types=(
            [pltpu.VMEM((_CHUNK,), jnp.int32)] * 4
            + [pltpu.VMEM((_CHUNK, msg), jnp.float32)] * 6
            + [pltpu.VMEM((n_nodes_pad // _NS, msg), jnp.float32),
               pltpu.VMEM_SHARED((n_nodes_pad, msg), jnp.float32)]
            + [pltpu.SemaphoreType.DMA] * 10
        ),
    )
    partials = sc_fn(p0, p1, e_pre, n0, n1)

    grid2 = 10
    nblk2 = n_nodes // grid2
    out = pl.pallas_call(
        _tc_post_body,
        grid=(grid2,),
        in_specs=[
            pl.BlockSpec((nblk2, d_feat), lambda i: (i, 0)),
            pl.BlockSpec((nblk2, msg), lambda i: (i, 0)),
            pl.BlockSpec((nblk2, msg), lambda i: (i, 0)),
            pl.BlockSpec((d_feat, d_feat), lambda i: (0, 0)),
            pl.BlockSpec((msg, d_feat), lambda i: (0, 0)),
            pl.BlockSpec((1, d_feat), lambda i: (0, 0)),
        ],
        out_specs=pl.BlockSpec((nblk2, d_feat), lambda i: (i, 0)),
        out_shape=jax.ShapeDtypeStruct((n_nodes, d_feat), jnp.float32),
    )(node_features, partials[0], partials[1], W2[:d_feat], W2[d_feat:], b2r)

    return out
